# Initial kernel scaffold; baseline (speedup 1.0000x reference)
import jax
import jax.numpy as jnp
from jax.experimental import pallas as pl


def kernel(input):
    raise NotImplementedError("write your pallas kernel here")


# speedup vs baseline: 3.6750x; 3.6750x over previous
import functools

import numpy as np
import jax
import jax.numpy as jnp
from jax import lax
from jax.experimental import pallas as pl
from jax.experimental.pallas import tpu as pltpu
from jax.experimental.pallas import tpu_sc as plsc

_IMAGE = 200
_PC = 50
_NPTS = _PC * _PC
_LANES = 16
_PAD = 2512
_NCHUNK = _PAD // _LANES
_FRAME = _IMAGE * _IMAGE


def _build_tables():
    x1 = np.linspace(0.0, 1.0, _PC).astype(np.float32)
    xpc = np.tile(x1, _PC)
    ypc = np.repeat(x1, _PC)
    gx = xpc * np.float32(_IMAGE - 1)
    gy = ypc * np.float32(_IMAGE - 1)
    ix0 = np.clip(np.floor(gx).astype(np.int32), 0, _IMAGE - 1)
    iy0 = np.clip(np.floor(gy).astype(np.int32), 0, _IMAGE - 1)
    ix1 = np.clip(ix0 + 1, 0, _IMAGE - 1)
    iy1 = np.clip(iy0 + 1, 0, _IMAGE - 1)
    wx = gx - ix0.astype(np.float32)
    wy = gy - iy0.astype(np.float32)
    idx = np.zeros((4, _PAD), np.int32)
    wts = np.zeros((4, _PAD), np.float32)
    idx[0, :_NPTS] = iy0 * _IMAGE + ix0
    idx[1, :_NPTS] = iy0 * _IMAGE + ix1
    idx[2, :_NPTS] = iy1 * _IMAGE + ix0
    idx[3, :_NPTS] = iy1 * _IMAGE + ix1
    wts[0, :_NPTS] = (1.0 - wx) * (1.0 - wy)
    wts[1, :_NPTS] = wx * (1.0 - wy)
    wts[2, :_NPTS] = (1.0 - wx) * wy
    wts[3, :_NPTS] = wx * wy
    return idx, wts


_IDX_TABLE, _WTS_TABLE = _build_tables()

_NC, _NS = 2, 16
_NW = _NC * _NS


def _make_sampler(n_frames):
    frames_per_w = n_frames // _NW
    mesh = plsc.VectorSubcoreMesh(core_axis_name="c", subcore_axis_name="s")

    @functools.partial(
        pl.kernel,
        mesh=mesh,
        out_type=jax.ShapeDtypeStruct((n_frames, _PAD), jnp.float32),
        compiler_params=pltpu.CompilerParams(needs_layout_passes=False),
        scratch_types=[
            pltpu.VMEM((4, _PAD), jnp.int32),
            pltpu.VMEM((4, _PAD), jnp.float32),
            pltpu.VMEM((_FRAME,), jnp.float32),
            pltpu.VMEM((_FRAME,), jnp.float32),
            pltpu.VMEM((_PAD,), jnp.float32),
            pltpu.SemaphoreType.DMA,
            pltpu.SemaphoreType.DMA,
        ],
    )
    def sampler(frames_hbm, idx_hbm, wts_hbm, out_hbm,
                idx_v, wts_v, fbuf0, fbuf1, ovec, sem0, sem1):
        wid = lax.axis_index("s") * _NC + lax.axis_index("c")
        base = wid * frames_per_w
        pltpu.sync_copy(idx_hbm, idx_v)
        pltpu.sync_copy(wts_hbm, wts_v)
        bufs = (fbuf0, fbuf1)
        sems = (sem0, sem1)
        nxt = pltpu.async_copy(frames_hbm.at[base], fbuf0, sem0)
        for k in range(frames_per_w):
            cur_buf = bufs[k % 2]
            cur_cp = nxt
            if k + 1 < frames_per_w:
                nxt = pltpu.async_copy(
                    frames_hbm.at[base + k + 1], bufs[(k + 1) % 2],
                    sems[(k + 1) % 2])
            cur_cp.wait()

            def chunk(i, carry):
                sl = pl.ds(i * _LANES, _LANES)
                v0 = plsc.load_gather(cur_buf, [idx_v[0, sl]])
                v1 = plsc.load_gather(cur_buf, [idx_v[1, sl]])
                v2 = plsc.load_gather(cur_buf, [idx_v[2, sl]])
                v3 = plsc.load_gather(cur_buf, [idx_v[3, sl]])
                ovec[sl] = (v0 * wts_v[0, sl] + v1 * wts_v[1, sl]
                            + v2 * wts_v[2, sl] + v3 * wts_v[3, sl])
                return carry

            lax.fori_loop(0, _NCHUNK, chunk, 0)
            pltpu.sync_copy(ovec, out_hbm.at[base + k])

    return sampler


def kernel(input):
    B, T, C, H, W = input.shape
    n_frames = B * T * C
    frames = input.reshape(n_frames, H * W)
    sampler = _make_sampler(n_frames)
    out = sampler(frames, jnp.asarray(_IDX_TABLE), jnp.asarray(_WTS_TABLE))
    res = out[:, :_NPTS].reshape(B, T, C, _NPTS)
    return (res, res)


# speedup vs baseline: 3.8206x; 1.0396x over previous
import functools

import numpy as np
import jax
import jax.numpy as jnp
from jax import lax
from jax.experimental import pallas as pl
from jax.experimental.pallas import tpu as pltpu
from jax.experimental.pallas import tpu_sc as plsc

_IMAGE = 200
_PC = 50
_NPTS = _PC * _PC
_LANES = 16
_PAD = 2560
_NCHUNK = _PAD // _LANES
_FRAME = _IMAGE * _IMAGE


def _build_tables():
    x1 = np.linspace(0.0, 1.0, _PC).astype(np.float32)
    xpc = np.tile(x1, _PC)
    ypc = np.repeat(x1, _PC)
    gx = xpc * np.float32(_IMAGE - 1)
    gy = ypc * np.float32(_IMAGE - 1)
    ix0 = np.clip(np.floor(gx).astype(np.int32), 0, _IMAGE - 1)
    iy0 = np.clip(np.floor(gy).astype(np.int32), 0, _IMAGE - 1)
    ix1 = np.clip(ix0 + 1, 0, _IMAGE - 1)
    iy1 = np.clip(iy0 + 1, 0, _IMAGE - 1)
    wx = gx - ix0.astype(np.float32)
    wy = gy - iy0.astype(np.float32)
    idx = np.zeros((4, _PAD), np.int32)
    wts = np.zeros((4, _PAD), np.float32)
    idx[0, :_NPTS] = iy0 * _IMAGE + ix0
    idx[1, :_NPTS] = iy0 * _IMAGE + ix1
    idx[2, :_NPTS] = iy1 * _IMAGE + ix0
    idx[3, :_NPTS] = iy1 * _IMAGE + ix1
    wts[0, :_NPTS] = (1.0 - wx) * (1.0 - wy)
    wts[1, :_NPTS] = wx * (1.0 - wy)
    wts[2, :_NPTS] = (1.0 - wx) * wy
    wts[3, :_NPTS] = wx * wy
    return idx, wts


_IDX_TABLE, _WTS_TABLE = _build_tables()

_NC, _NS = 2, 16
_NW = _NC * _NS


def _make_sampler(n_frames):
    frames_per_w = n_frames // _NW
    mesh = plsc.VectorSubcoreMesh(core_axis_name="c", subcore_axis_name="s")

    @functools.partial(
        pl.kernel,
        mesh=mesh,
        out_type=jax.ShapeDtypeStruct((n_frames, _PAD), jnp.float32),
        compiler_params=pltpu.CompilerParams(needs_layout_passes=False),
        scratch_types=[
            pltpu.VMEM((4, _PAD), jnp.int32),
            pltpu.VMEM((4, _PAD), jnp.float32),
            pltpu.VMEM((_FRAME,), jnp.float32),
            pltpu.VMEM((_FRAME,), jnp.float32),
            pltpu.VMEM((_PAD,), jnp.float32),
            pltpu.SemaphoreType.DMA,
            pltpu.SemaphoreType.DMA,
        ],
    )
    def sampler(frames_hbm, idx_hbm, wts_hbm, out_hbm,
                idx_v, wts_v, fbuf0, fbuf1, ovec, sem0, sem1):
        wid = lax.axis_index("s") * _NC + lax.axis_index("c")
        base = wid * frames_per_w
        pltpu.sync_copy(idx_hbm, idx_v)
        pltpu.sync_copy(wts_hbm, wts_v)
        bufs = (fbuf0, fbuf1)
        sems = (sem0, sem1)
        nxt = pltpu.async_copy(frames_hbm.at[base], fbuf0, sem0)
        for k in range(frames_per_w):
            cur_buf = bufs[k % 2]
            cur_cp = nxt
            if k + 1 < frames_per_w:
                nxt = pltpu.async_copy(
                    frames_hbm.at[base + k + 1], bufs[(k + 1) % 2],
                    sems[(k + 1) % 2])
            cur_cp.wait()

            @plsc.parallel_loop(0, _NCHUNK, unroll=8)
            def _chunk(i):
                sl = pl.ds(i * _LANES, _LANES)
                v0 = plsc.load_gather(cur_buf, [idx_v[0, sl]])
                v1 = plsc.load_gather(cur_buf, [idx_v[1, sl]])
                v2 = plsc.load_gather(cur_buf, [idx_v[2, sl]])
                v3 = plsc.load_gather(cur_buf, [idx_v[3, sl]])
                ovec[sl] = (v0 * wts_v[0, sl] + v1 * wts_v[1, sl]
                            + v2 * wts_v[2, sl] + v3 * wts_v[3, sl])

            pltpu.sync_copy(ovec, out_hbm.at[base + k])

    return sampler


def kernel(input):
    B, T, C, H, W = input.shape
    n_frames = B * T * C
    frames = input.reshape(n_frames, H * W)
    sampler = _make_sampler(n_frames)
    out = sampler(frames, jnp.asarray(_IDX_TABLE), jnp.asarray(_WTS_TABLE))
    res = out[:, :_NPTS].reshape(B, T, C, _NPTS)
    return (res, res)


# speedup vs baseline: 5.8615x; 1.5342x over previous
import functools

import numpy as np
import jax
import jax.numpy as jnp
from jax import lax
from jax.experimental import pallas as pl
from jax.experimental.pallas import tpu as pltpu
from jax.experimental.pallas import tpu_sc as plsc

_IMAGE = 200
_PC = 50
_NPTS = _PC * _PC
_LANES = 16
_PAD = 2560
_NCHUNK = _PAD // _LANES
_OPAD = 2512
_NC, _NS = 2, 16
_NW = _NC * _NS


def _build_tables():
    x1 = np.linspace(0.0, 1.0, _PC).astype(np.float32)
    xpc = np.tile(x1, _PC)
    ypc = np.repeat(x1, _PC)
    gx = xpc * np.float32(_IMAGE - 1)
    gy = ypc * np.float32(_IMAGE - 1)
    ix0 = np.clip(np.floor(gx).astype(np.int32), 0, _IMAGE - 1)
    iy0 = np.clip(np.floor(gy).astype(np.int32), 0, _IMAGE - 1)
    ix1 = np.clip(ix0 + 1, 0, _IMAGE - 1)
    iy1 = np.clip(iy0 + 1, 0, _IMAGE - 1)
    wx = gx - ix0.astype(np.float32)
    wy = gy - iy0.astype(np.float32)
    idx = np.zeros((4, _PAD), np.int32)
    wts = np.zeros((4, _PAD), np.float32)
    idx[0, :_NPTS] = iy0
    idx[1, :_NPTS] = iy1
    idx[2, :_NPTS] = ix0
    idx[3, :_NPTS] = ix1
    wts[0, :_NPTS] = (1.0 - wx) * (1.0 - wy)
    wts[1, :_NPTS] = wx * (1.0 - wy)
    wts[2, :_NPTS] = (1.0 - wx) * wy
    wts[3, :_NPTS] = wx * wy
    return idx, wts


_IDX_TABLE, _WTS_TABLE = _build_tables()


def _make_sampler(n_frames):
    frames_per_w = n_frames // _NW
    mesh = plsc.VectorSubcoreMesh(core_axis_name="c", subcore_axis_name="s")

    @functools.partial(
        pl.kernel,
        mesh=mesh,
        out_type=jax.ShapeDtypeStruct((n_frames, _PAD), jnp.float32),
        compiler_params=pltpu.CompilerParams(needs_layout_passes=False),
        scratch_types=[
            pltpu.VMEM((4, _PAD), jnp.int32),
            pltpu.VMEM((4, _PAD), jnp.float32),
            pltpu.VMEM((_IMAGE, _IMAGE), jnp.float32),
            pltpu.VMEM((_IMAGE, _IMAGE), jnp.float32),
            pltpu.VMEM((_PAD,), jnp.float32),
            pltpu.SemaphoreType.DMA,
            pltpu.SemaphoreType.DMA,
        ],
    )
    def sampler(frames_hbm, idx_hbm, wts_hbm, out_hbm,
                idx_v, wts_v, fbuf0, fbuf1, ovec, sem0, sem1):
        wid = lax.axis_index("s") * _NC + lax.axis_index("c")
        base = wid * frames_per_w
        pltpu.sync_copy(idx_hbm, idx_v)
        pltpu.sync_copy(wts_hbm, wts_v)
        bufs = (fbuf0, fbuf1)
        sems = (sem0, sem1)
        nxt = pltpu.async_copy(frames_hbm.at[base], fbuf0, sem0)
        for k in range(frames_per_w):
            cur_buf = bufs[k % 2]
            cur_cp = nxt
            if k + 1 < frames_per_w:
                nxt = pltpu.async_copy(
                    frames_hbm.at[base + k + 1], bufs[(k + 1) % 2],
                    sems[(k + 1) % 2])
            cur_cp.wait()

            @plsc.parallel_loop(0, _NCHUNK, unroll=8)
            def _chunk(i):
                sl = pl.ds(i * _LANES, _LANES)
                iy0 = idx_v[0, sl]
                iy1 = idx_v[1, sl]
                ix0 = idx_v[2, sl]
                ix1 = idx_v[3, sl]
                v0 = plsc.load_gather(cur_buf, [iy0, ix0])
                v1 = plsc.load_gather(cur_buf, [iy0, ix1])
                v2 = plsc.load_gather(cur_buf, [iy1, ix0])
                v3 = plsc.load_gather(cur_buf, [iy1, ix1])
                ovec[sl] = (v0 * wts_v[0, sl] + v1 * wts_v[1, sl]
                            + v2 * wts_v[2, sl] + v3 * wts_v[3, sl])

            pltpu.sync_copy(ovec, out_hbm.at[base + k])

    return sampler


def kernel(input):
    B, T, C, H, W = input.shape
    n_frames = B * T * C
    frames = input.reshape(n_frames, H, W)
    sampler = _make_sampler(n_frames)
    out = sampler(frames, jnp.asarray(_IDX_TABLE), jnp.asarray(_WTS_TABLE))
    res = out[:, :_NPTS].reshape(B, T, C, _NPTS)
    return (res, res)
